# baseline (speedup 1.0000x reference)
import jax
import jax.numpy as jnp
from jax.experimental import pallas as pl
from jax.experimental.pallas import tpu as pltpu

B, T, D = 256, 200, 128

def _body(s0, s1, s2, o_ref, buf, sems):
    c0 = pltpu.make_async_copy(s0, buf, sems.at[0])
    c1 = pltpu.make_async_copy(s1, buf, sems.at[1])
    c2 = pltpu.make_async_copy(s2, buf, sems.at[2])
    c0.start(); c1.start(); c2.start()
    c0.wait(); c1.wait(); c2.wait()
    o_ref[...] = buf[0, :8, :]

def kernel(seg0, seg1, seg2, sp_table, num_cls):
    return pl.pallas_call(
        _body,
        in_specs=[pl.BlockSpec(memory_space=pl.ANY)] * 3,
        out_specs=pl.BlockSpec(memory_space=pltpu.VMEM),
        out_shape=jax.ShapeDtypeStruct((8, D), jnp.float32),
        scratch_shapes=[
            pltpu.VMEM((B, T, D), jnp.float32),
            pltpu.SemaphoreType.DMA((3,)),
        ],
    )(seg0, seg1, seg2)
